# baseline (speedup 1.0000x reference)
import jax
import jax.numpy as jnp
from jax.experimental import pallas as pl
from jax.experimental.pallas import tpu as pltpu

B, T, HID = 2, 256, 1024
H, DK, M, TOPK = 4, 64, 8, 2
KD = H * DK
VD = 2 * KD
DV = VD // H
BT = B * T
SH = TOPK * H
SKD = TOPK * H * DK
NS = 6 * SH

_F32 = jnp.float32


def _silu(x):
    return x * jax.nn.sigmoid(x)


def _dense_stage(x_ref, gate_w_ref, q_w_ref, k_w_ref, v_w_ref, b_w_ref,
                 a_w_ref, g_w_ref, A_log_ref, dt_bias_ref,
                 kqbd_ref, vbs_ref, gl_ref,
                 sel_ref, scalT_ref, dec_ref):
    x2 = x_ref[...].reshape(BT, HID)

    logits = jnp.dot(x2, gate_w_ref[...], preferred_element_type=_F32)
    mx = jnp.max(logits, axis=1, keepdims=True)
    e = jnp.exp(logits - mx)
    s = e / jnp.sum(e, axis=1, keepdims=True)
    lane = jax.lax.broadcasted_iota(jnp.int32, (BT, M), 1)
    m1 = jnp.max(s, axis=1, keepdims=True)
    i1 = jnp.min(jnp.where(s == m1, lane, M), axis=1, keepdims=True)
    s2 = jnp.where(lane == i1, -1.0, s)
    m2 = jnp.max(s2, axis=1, keepdims=True)
    i2 = jnp.min(jnp.where(s2 == m2, lane, M), axis=1, keepdims=True)
    denom = m1 + m2
    sel_ref[...] = jnp.concatenate([i1, i2], axis=1).reshape(B, T, TOPK)
    rw = [m1 / denom, m2 / denom]

    beta = jax.nn.sigmoid(jnp.dot(x2, b_w_ref[...], preferred_element_type=_F32))
    a = jnp.dot(x2, a_w_ref[...], preferred_element_type=_F32) + dt_bias_ref[...]
    sp = jnp.maximum(a, 0.0) + jnp.log1p(jnp.exp(-jnp.abs(a)))
    dec = jnp.exp(-jnp.exp(A_log_ref[...]) * sp)
    dec_ref[...] = dec.reshape(B, T, H)
    bd = beta * dec

    qs = _silu(jnp.dot(x2, q_w_ref[...], preferred_element_type=_F32))
    ks = _silu(jnp.dot(x2, k_w_ref[...], preferred_element_type=_F32))
    vs = _silu(jnp.dot(x2, v_w_ref[...], preferred_element_type=_F32))
    gs = jnp.dot(x2, g_w_ref[...], preferred_element_type=_F32)
    scale = DK ** -0.5

    kqbd_ref[...] = jnp.zeros((B, T, 2 * SH, SKD), dtype=jnp.bfloat16)
    qk_cols = []
    kk_cols = []
    qkp_cols = []
    for hh in range(H):
        qh = qs[:, hh * DK:(hh + 1) * DK]
        nq = jnp.sqrt(jnp.sum(qh * qh, axis=1, keepdims=True))
        qn = qh / jnp.maximum(nq, 1e-12) * scale
        kh = ks[:, hh * DK:(hh + 1) * DK]
        nk = jnp.sqrt(jnp.sum(kh * kh, axis=1, keepdims=True))
        kn = kh / jnp.maximum(nk, 1e-12)
        qk_cols.append(jnp.sum(qn * kn, axis=1, keepdims=True))
        knm1 = jnp.concatenate([jnp.zeros((1, DK), dtype=_F32), kn[:-1]], axis=0)
        kk_cols.append(jnp.sum(kn * knm1, axis=1, keepdims=True))
        qkp_cols.append(jnp.sum(qn * knm1, axis=1, keepdims=True))
        vb_h = beta[:, hh:hh + 1] * vs[:, hh * DV:(hh + 1) * DV]
        gl_ref[:, hh] = gs[:, hh * DV:(hh + 1) * DV].reshape(B, T, DV)
        for slot in range(TOPK):
            r = slot * H + hh
            c = slot * KD + hh * DK
            kqbd_ref[:, :, r, c:c + DK] = kn.astype(jnp.bfloat16).reshape(B, T, DK)
            kqbd_ref[:, :, SH + r, c:c + DK] = qn.astype(jnp.bfloat16).reshape(B, T, DK)
            vbs_ref[:, :, r, :] = vb_h.reshape(B, T, DV)

    cols = []
    for hh in range(H):
        cols.append(bd[:, hh:hh + 1])
    for hh in range(H):
        cols.append(bd[:, hh:hh + 1])
    for slot in range(TOPK):
        for hh in range(H):
            cols.append(rw[slot] * dec[:, hh:hh + 1])
    for slot in range(TOPK):
        for hh in range(H):
            cols.append(rw[slot] * qk_cols[hh])
    for _ in range(TOPK):
        for hh in range(H):
            cols.append(dec[:, hh:hh + 1])
    for _ in range(TOPK):
        for hh in range(H):
            cols.append(kk_cols[hh])
    for _ in range(TOPK):
        for hh in range(H):
            cols.append(qkp_cols[hh])
    scalT_ref[...] = jnp.concatenate(cols, axis=1).reshape(B, T, NS)


def _scan_stage(kqbd_ref, vbs_ref, sel_ref, scalT_ref, dec_ref,
                oc_ref, h0_ref, h1_ref):
    h0_ref[...] = jnp.zeros((M, KD, DV), dtype=_F32)
    h1_ref[...] = jnp.zeros((M, KD, DV), dtype=_F32)
    h_refs = (h0_ref, h1_ref)

    vnp0 = []
    for b in range(B):
        cols0 = jnp.transpose(scalT_ref[b, pl.ds(0, 1), :], (1, 0))
        wq8 = cols0[2 * SH:3 * SH]
        vnew8 = vbs_ref[b, pl.ds(0, 1)].reshape(SH, DV)
        o8 = wq8 * vnew8
        acc = o8[0:H] + o8[H:SH]
        oc_ref[b, pl.ds(0, 1)] = acc.reshape(1, H, DV)
        vnp0.append(vnew8)

    def step(t, vnp_pair):
        new_pair = []
        for b in range(B):
            h_ref = h_refs[b]
            vnp = vnp_pair[b]
            i0 = sel_ref[b, t, 0]
            i1 = sel_ref[b, t, 1]
            j0 = sel_ref[b, t - 1, 0]
            j1 = sel_ref[b, t - 1, 1]
            hp = jnp.concatenate([h_ref[i0], h_ref[i1]], axis=0)
            hpb = hp.astype(jnp.bfloat16)
            kqb = kqbd_ref[b, pl.ds(t, 1)].reshape(2 * SH, SKD)
            kbp = kqbd_ref[b, pl.ds(t - 1, 1)].reshape(2 * SH, SKD)[0:SH]
            vb8 = vbs_ref[b, pl.ds(t, 1)].reshape(SH, DV)
            cols2 = jnp.transpose(scalT_ref[b, pl.ds(t - 1, 2), :], (1, 0))
            bd8 = cols2[0:SH, 1:2]
            wd8 = cols2[SH:2 * SH, 1:2]
            wq8 = cols2[2 * SH:3 * SH, 1:2]
            dec8p = cols2[3 * SH:4 * SH, 0:1]
            kk8 = cols2[4 * SH:5 * SH, 1:2]
            qk8 = cols2[5 * SH:6 * SH, 1:2]
            pq = jax.lax.dot_general(kqb, hpb, (((1,), (0,)), ((), ())),
                                     preferred_element_type=_F32)
            p0 = pq[0:SH]
            q0 = pq[SH:2 * SH]
            outer = jax.lax.dot_general(kbp, vnp.astype(jnp.bfloat16),
                                        (((0,), (0,)), ((), ())),
                                        preferred_element_type=_F32)
            pred_h, qh_h = [], []
            for s, i_s in ((0, i0), (1, i1)):
                e0 = (i_s == j0).astype(_F32)
                e1 = (i_s == j1).astype(_F32)
                ts = e0 + e1
                alpha = dec8p[s * H:(s + 1) * H] * ts + (1.0 - ts)
                corr = e0 * vnp[0:H] + e1 * vnp[H:SH]
                pred_h.append(alpha * p0[s * H:(s + 1) * H]
                              + kk8[s * H:(s + 1) * H] * corr)
                qh_h.append(alpha * q0[s * H:(s + 1) * H]
                            + qk8[s * H:(s + 1) * H] * corr)
            pred = jnp.concatenate(pred_h, axis=0)
            qh = jnp.concatenate(qh_h, axis=0)
            vnew8 = vb8 - bd8 * pred
            o8 = wd8 * qh + wq8 * vnew8
            acc = o8[0:H] + o8[H:SH]
            oc_ref[b, pl.ds(t, 1)] = acc.reshape(1, H, DV)
            for s, j_s in ((0, j0), (1, j1)):
                slab = h_ref[j_s]
                base = s * KD
                for hh in range(H):
                    decp = cols2[3 * SH + hh:3 * SH + hh + 1, 0:1]
                    h_ref[j_s, hh * DK:(hh + 1) * DK] = (
                        slab[hh * DK:(hh + 1) * DK] * decp
                        + outer[base + hh * DK:base + (hh + 1) * DK])
            new_pair.append(vnew8)
        return tuple(new_pair)

    jax.lax.fori_loop(1, T, step, (vnp0[0], vnp0[1]), unroll=17)


def _out_stage(oc_ref, gl_ref, o_w_ref, onw_ref, out_ref):
    for b in range(B):
        acc = jnp.zeros((T, HID), dtype=_F32)
        for hh in range(H):
            y = oc_ref[b, :, hh, :]
            rms = jnp.sqrt(jnp.mean(y * y, axis=1, keepdims=True) + 1e-6)
            srow = (y / rms) * onw_ref[...] * jax.nn.sigmoid(gl_ref[b, hh])
            acc = acc + jnp.dot(srow, o_w_ref[hh * DV:(hh + 1) * DV, :],
                                preferred_element_type=_F32)
        out_ref[b] = acc


def _vmem():
    return pl.BlockSpec(memory_space=pltpu.VMEM)


def _smem():
    return pl.BlockSpec(memory_space=pltpu.SMEM)


@jax.jit
def kernel(x, gate_w, q_w, k_w, v_w, b_w, a_w, g_w, o_w, A_log, dt_bias,
           o_norm_weight):
    A_log2 = A_log.reshape(1, H)
    dt2 = dt_bias.reshape(1, H)
    onw2 = o_norm_weight.reshape(1, DV)

    kqbd, vbs, gl, sel, scalT, dec = pl.pallas_call(
        _dense_stage,
        in_specs=[_vmem()] * 10,
        out_specs=(_vmem(),) * 6,
        out_shape=(
            jax.ShapeDtypeStruct((B, T, 2 * SH, SKD), jnp.bfloat16),
            jax.ShapeDtypeStruct((B, T, SH, DV), _F32),
            jax.ShapeDtypeStruct((B, H, T, DV), _F32),
            jax.ShapeDtypeStruct((B, T, TOPK), jnp.int32),
            jax.ShapeDtypeStruct((B, T, NS), _F32),
            jax.ShapeDtypeStruct((B, T, H), _F32),
        ),
    )(x, gate_w, q_w, k_w, v_w, b_w, a_w, g_w, A_log2, dt2)

    oc = pl.pallas_call(
        _scan_stage,
        in_specs=[_vmem(), _vmem(), _smem(), _vmem(), _vmem()],
        out_specs=_vmem(),
        out_shape=jax.ShapeDtypeStruct((B, T, H, DV), _F32),
        scratch_shapes=[pltpu.VMEM((M, KD, DV), _F32),
                        pltpu.VMEM((M, KD, DV), _F32)],
    )(kqbd, vbs, sel, scalT, dec)

    out = pl.pallas_call(
        _out_stage,
        in_specs=[_vmem(), _vmem(), _vmem(), _vmem()],
        out_specs=_vmem(),
        out_shape=jax.ShapeDtypeStruct((B, T, HID), _F32),
    )(oc, gl, o_w, onw2)
    return out


# speedup vs baseline: 1.0014x; 1.0014x over previous
import jax
import jax.numpy as jnp
from jax.experimental import pallas as pl
from jax.experimental.pallas import tpu as pltpu

B, T, HID = 2, 256, 1024
H, DK, M, TOPK = 4, 64, 8, 2
KD = H * DK
VD = 2 * KD
DV = VD // H
BT = B * T
SH = TOPK * H
SKD = TOPK * H * DK
NS = 6 * SH

_F32 = jnp.float32


def _silu(x):
    return x * jax.nn.sigmoid(x)


def _dense_stage(x_ref, gate_w_ref, q_w_ref, k_w_ref, v_w_ref, b_w_ref,
                 a_w_ref, g_w_ref, A_log_ref, dt_bias_ref,
                 kqbd_ref, vbs_ref, gl_ref,
                 sel_ref, scalT_ref, dec_ref):
    x2 = x_ref[...].reshape(BT, HID)

    logits = jnp.dot(x2, gate_w_ref[...], preferred_element_type=_F32)
    mx = jnp.max(logits, axis=1, keepdims=True)
    e = jnp.exp(logits - mx)
    s = e / jnp.sum(e, axis=1, keepdims=True)
    lane = jax.lax.broadcasted_iota(jnp.int32, (BT, M), 1)
    m1 = jnp.max(s, axis=1, keepdims=True)
    i1 = jnp.min(jnp.where(s == m1, lane, M), axis=1, keepdims=True)
    s2 = jnp.where(lane == i1, -1.0, s)
    m2 = jnp.max(s2, axis=1, keepdims=True)
    i2 = jnp.min(jnp.where(s2 == m2, lane, M), axis=1, keepdims=True)
    denom = m1 + m2
    sel_ref[...] = jnp.concatenate([i1, i2], axis=1).reshape(B, T, TOPK)
    rw = [m1 / denom, m2 / denom]

    beta = jax.nn.sigmoid(jnp.dot(x2, b_w_ref[...], preferred_element_type=_F32))
    a = jnp.dot(x2, a_w_ref[...], preferred_element_type=_F32) + dt_bias_ref[...]
    sp = jnp.maximum(a, 0.0) + jnp.log1p(jnp.exp(-jnp.abs(a)))
    dec = jnp.exp(-jnp.exp(A_log_ref[...]) * sp)
    dec_ref[...] = dec.reshape(B, T, H)
    bd = beta * dec

    qs = _silu(jnp.dot(x2, q_w_ref[...], preferred_element_type=_F32))
    ks = _silu(jnp.dot(x2, k_w_ref[...], preferred_element_type=_F32))
    vs = _silu(jnp.dot(x2, v_w_ref[...], preferred_element_type=_F32))
    gs = jnp.dot(x2, g_w_ref[...], preferred_element_type=_F32)
    scale = DK ** -0.5

    kqbd_ref[...] = jnp.zeros((B, T, 2 * SH, SKD), dtype=jnp.bfloat16)
    qk_cols = []
    kk_cols = []
    qkp_cols = []
    for hh in range(H):
        qh = qs[:, hh * DK:(hh + 1) * DK]
        nq = jnp.sqrt(jnp.sum(qh * qh, axis=1, keepdims=True))
        qn = qh / jnp.maximum(nq, 1e-12) * scale
        kh = ks[:, hh * DK:(hh + 1) * DK]
        nk = jnp.sqrt(jnp.sum(kh * kh, axis=1, keepdims=True))
        kn = kh / jnp.maximum(nk, 1e-12)
        qk_cols.append(jnp.sum(qn * kn, axis=1, keepdims=True))
        knm1 = jnp.concatenate([jnp.zeros((1, DK), dtype=_F32), kn[:-1]], axis=0)
        kk_cols.append(jnp.sum(kn * knm1, axis=1, keepdims=True))
        qkp_cols.append(jnp.sum(qn * knm1, axis=1, keepdims=True))
        vb_h = beta[:, hh:hh + 1] * vs[:, hh * DV:(hh + 1) * DV]
        gl_ref[:, hh] = gs[:, hh * DV:(hh + 1) * DV].reshape(B, T, DV)
        for slot in range(TOPK):
            r = slot * H + hh
            c = slot * KD + hh * DK
            kqbd_ref[:, :, r, c:c + DK] = kn.astype(jnp.bfloat16).reshape(B, T, DK)
            kqbd_ref[:, :, SH + r, c:c + DK] = qn.astype(jnp.bfloat16).reshape(B, T, DK)
            vbs_ref[:, :, r, :] = vb_h.reshape(B, T, DV)

    cols = []
    for hh in range(H):
        cols.append(bd[:, hh:hh + 1])
    for hh in range(H):
        cols.append(bd[:, hh:hh + 1])
    for slot in range(TOPK):
        for hh in range(H):
            cols.append(rw[slot] * dec[:, hh:hh + 1])
    for slot in range(TOPK):
        for hh in range(H):
            cols.append(rw[slot] * qk_cols[hh])
    for _ in range(TOPK):
        for hh in range(H):
            cols.append(dec[:, hh:hh + 1])
    for _ in range(TOPK):
        for hh in range(H):
            cols.append(kk_cols[hh])
    for _ in range(TOPK):
        for hh in range(H):
            cols.append(qkp_cols[hh])
    scalT_ref[...] = jnp.concatenate(cols, axis=1).reshape(B, T, NS)


def _scan_stage(kqbd_ref, vbs_ref, sel_ref, scalT_ref, dec_ref,
                oc_ref, h0_ref, h1_ref):
    h0_ref[...] = jnp.zeros((M, KD, DV), dtype=_F32)
    h1_ref[...] = jnp.zeros((M, KD, DV), dtype=_F32)
    h_refs = (h0_ref, h1_ref)

    vnp0 = []
    for b in range(B):
        cols0 = jnp.transpose(scalT_ref[b, pl.ds(0, 1), :], (1, 0))
        wq8 = cols0[2 * SH:3 * SH]
        vnew8 = vbs_ref[b, pl.ds(0, 1)].reshape(SH, DV)
        o8 = wq8 * vnew8
        acc = o8[0:H] + o8[H:SH]
        oc_ref[b, pl.ds(0, 1)] = acc.reshape(1, H, DV)
        vnp0.append(vnew8)

    def step(t, vnp_pair):
        new_pair = []
        for b in range(B):
            h_ref = h_refs[b]
            vnp = vnp_pair[b]
            i0 = sel_ref[b, t, 0]
            i1 = sel_ref[b, t, 1]
            j0 = sel_ref[b, t - 1, 0]
            j1 = sel_ref[b, t - 1, 1]
            hp = jnp.concatenate([h_ref[i0], h_ref[i1]], axis=0)
            hpb = hp.astype(jnp.bfloat16)
            kqb = kqbd_ref[b, pl.ds(t, 1)].reshape(2 * SH, SKD)
            kbp = kqbd_ref[b, pl.ds(t - 1, 1)].reshape(2 * SH, SKD)[0:SH]
            vb8 = vbs_ref[b, pl.ds(t, 1)].reshape(SH, DV)
            cols2 = jnp.transpose(scalT_ref[b, pl.ds(t - 1, 2), :], (1, 0))
            bd8 = cols2[0:SH, 1:2]
            wd8 = cols2[SH:2 * SH, 1:2]
            wq8 = cols2[2 * SH:3 * SH, 1:2]
            dec8p = cols2[3 * SH:4 * SH, 0:1]
            kk8 = cols2[4 * SH:5 * SH, 1:2]
            qk8 = cols2[5 * SH:6 * SH, 1:2]
            pq = jax.lax.dot_general(kqb, hpb, (((1,), (0,)), ((), ())),
                                     preferred_element_type=_F32)
            p0 = pq[0:SH]
            q0 = pq[SH:2 * SH]
            outer = jax.lax.dot_general(kbp, vnp.astype(jnp.bfloat16),
                                        (((0,), (0,)), ((), ())),
                                        preferred_element_type=_F32)
            pred_h, qh_h = [], []
            for s, i_s in ((0, i0), (1, i1)):
                e0 = (i_s == j0).astype(_F32)
                e1 = (i_s == j1).astype(_F32)
                ts = e0 + e1
                alpha = dec8p[s * H:(s + 1) * H] * ts + (1.0 - ts)
                corr = e0 * vnp[0:H] + e1 * vnp[H:SH]
                pred_h.append(alpha * p0[s * H:(s + 1) * H]
                              + kk8[s * H:(s + 1) * H] * corr)
                qh_h.append(alpha * q0[s * H:(s + 1) * H]
                            + qk8[s * H:(s + 1) * H] * corr)
            pred = jnp.concatenate(pred_h, axis=0)
            qh = jnp.concatenate(qh_h, axis=0)
            vnew8 = vb8 - bd8 * pred
            o8 = wd8 * qh + wq8 * vnew8
            acc = o8[0:H] + o8[H:SH]
            oc_ref[b, pl.ds(t, 1)] = acc.reshape(1, H, DV)
            for s, j_s in ((0, j0), (1, j1)):
                slab = h_ref[j_s]
                base = s * KD
                for hh in range(H):
                    decp = cols2[3 * SH + hh:3 * SH + hh + 1, 0:1]
                    h_ref[j_s, hh * DK:(hh + 1) * DK] = (
                        slab[hh * DK:(hh + 1) * DK] * decp
                        + outer[base + hh * DK:base + (hh + 1) * DK])
            new_pair.append(vnew8)
        return tuple(new_pair)

    jax.lax.fori_loop(1, T, step, (vnp0[0], vnp0[1]), unroll=8)


def _out_stage(oc_ref, gl_ref, o_w_ref, onw_ref, out_ref):
    for b in range(B):
        acc = jnp.zeros((T, HID), dtype=_F32)
        for hh in range(H):
            y = oc_ref[b, :, hh, :]
            rms = jnp.sqrt(jnp.mean(y * y, axis=1, keepdims=True) + 1e-6)
            srow = (y / rms) * onw_ref[...] * jax.nn.sigmoid(gl_ref[b, hh])
            acc = acc + jnp.dot(srow, o_w_ref[hh * DV:(hh + 1) * DV, :],
                                preferred_element_type=_F32)
        out_ref[b] = acc


def _vmem():
    return pl.BlockSpec(memory_space=pltpu.VMEM)


def _smem():
    return pl.BlockSpec(memory_space=pltpu.SMEM)


@jax.jit
def kernel(x, gate_w, q_w, k_w, v_w, b_w, a_w, g_w, o_w, A_log, dt_bias,
           o_norm_weight):
    A_log2 = A_log.reshape(1, H)
    dt2 = dt_bias.reshape(1, H)
    onw2 = o_norm_weight.reshape(1, DV)

    kqbd, vbs, gl, sel, scalT, dec = pl.pallas_call(
        _dense_stage,
        in_specs=[_vmem()] * 10,
        out_specs=(_vmem(),) * 6,
        out_shape=(
            jax.ShapeDtypeStruct((B, T, 2 * SH, SKD), jnp.bfloat16),
            jax.ShapeDtypeStruct((B, T, SH, DV), _F32),
            jax.ShapeDtypeStruct((B, H, T, DV), _F32),
            jax.ShapeDtypeStruct((B, T, TOPK), jnp.int32),
            jax.ShapeDtypeStruct((B, T, NS), _F32),
            jax.ShapeDtypeStruct((B, T, H), _F32),
        ),
    )(x, gate_w, q_w, k_w, v_w, b_w, a_w, g_w, A_log2, dt2)

    oc = pl.pallas_call(
        _scan_stage,
        in_specs=[_vmem(), _vmem(), _smem(), _vmem(), _vmem()],
        out_specs=_vmem(),
        out_shape=jax.ShapeDtypeStruct((B, T, H, DV), _F32),
        scratch_shapes=[pltpu.VMEM((M, KD, DV), _F32),
                        pltpu.VMEM((M, KD, DV), _F32)],
    )(kqbd, vbs, sel, scalT, dec)

    out = pl.pallas_call(
        _out_stage,
        in_specs=[_vmem(), _vmem(), _vmem(), _vmem()],
        out_specs=_vmem(),
        out_shape=jax.ShapeDtypeStruct((B, T, HID), _F32),
    )(oc, gl, o_w, onw2)
    return out
